# baseline (speedup 1.0000x reference)
import functools

import jax
import jax.numpy as jnp
from jax import lax
from jax.experimental import pallas as pl
from jax.experimental.pallas import tpu as pltpu
from jax.experimental.pallas import tpu_sc as plsc

D = 64
BLK = 128


@functools.lru_cache(maxsize=None)
def _make_sc_kernel(B, L, NC, NS):
    NW = NC * NS
    KB = B // BLK
    assert KB == NW
    NBUF = 3
    n_groups = L // NBUF
    tail = L - n_groups * NBUF
    mesh = plsc.VectorSubcoreMesh(core_axis_name="c", subcore_axis_name="s")

    @functools.partial(
        pl.kernel,
        out_type=jax.ShapeDtypeStruct((L, D // 8, KB, 8, BLK), jnp.float32),
        mesh=mesh,
        compiler_params=pltpu.CompilerParams(use_tc_tiling_on_sc=False,
                                             needs_layout_passes=False),
        scratch_types=[
            pltpu.VMEM((L, BLK), jnp.int32),
            pltpu.VMEM((L, BLK), jnp.int32),
            pltpu.VMEM((L, BLK), jnp.int32),
            [pltpu.VMEM((BLK, D), jnp.float32)] * NBUF,
            [pltpu.VMEM((D // 8, 8, BLK), jnp.float32)] * NBUF,
            [pltpu.SemaphoreType.DMA] * NBUF,
            [pltpu.SemaphoreType.DMA] * NBUF,
            [pltpu.SemaphoreType.DMA] * NBUF,
        ],
    )
    def body(data_h, shift_h, total_h, wd_h, ws_h, wt_h, out_h,
             idx_d, idx_s, idx_t, accs, tbufs, gsems, asems, ssems):
        wid = lax.axis_index("s") * NC + lax.axis_index("c")
        pltpu.sync_copy(data_h.at[wid], idx_d)
        pltpu.sync_copy(shift_h.at[wid], idx_s)
        pltpu.sync_copy(total_h.at[wid], idx_t)

        lane = lax.iota(jnp.int32, 16)
        g_c = [(lane + 16 * m) // 8 for m in range(D // 16)]
        r_c = [lane % 8 for _ in range(D // 16)]

        def transpose_unit(s):
            @plsc.parallel_loop(0, BLK, unroll=4)
            def b_body(b):
                bsplat = jnp.full((16,), b, jnp.int32)
                for m in range(D // 16):
                    v = accs[s][b, pl.ds(16 * m, 16)]
                    plsc.store_scatter(tbufs[s], [g_c[m], r_c[m], bsplat], v)

        def unit_stage1(s, l, first):
            @pl.when(jnp.logical_not(first))
            def _wait_prev_store():
                pltpu.make_async_copy(tbufs[s], out_h.at[l - NBUF, :, wid],
                                      ssems[s]).wait()
            pltpu.async_copy(wd_h.at[idx_d.at[l]], accs[s], gsems[s])

        def unit_stage2(s, l):
            pltpu.make_async_copy(wd_h.at[idx_d.at[l]], accs[s],
                                  gsems[s]).wait()
            pltpu.async_copy(ws_h.at[idx_s.at[l]], accs[s], asems[s],
                             add=True)
            pltpu.async_copy(wt_h.at[idx_t.at[l]], accs[s], asems[s],
                             add=True)

        def unit_stage3(s, l):
            add_cp = pltpu.make_async_copy(ws_h.at[idx_s.at[l]], accs[s],
                                           asems[s])
            add_cp.wait()
            add_cp.wait()
            transpose_unit(s)
            pltpu.async_copy(tbufs[s], out_h.at[l, :, wid], ssems[s])

        def group_body(g, carry):
            for s in range(NBUF):
                unit_stage1(s, g * NBUF + s, g == 0)
            for s in range(NBUF):
                unit_stage2(s, g * NBUF + s)
            for s in range(NBUF):
                unit_stage3(s, g * NBUF + s)
            return carry

        lax.fori_loop(0, n_groups, group_body, 0)
        for s in range(tail):
            l = n_groups * NBUF + s
            unit_stage1(s, l, False)
            unit_stage2(s, l)
            unit_stage3(s, l)
        for s in range(NBUF):
            l = (n_groups - 1) * NBUF + s
            if s < tail:
                l = n_groups * NBUF + s
            pltpu.make_async_copy(tbufs[s], out_h.at[l, :, wid],
                                  ssems[s]).wait()

    return body


def kernel(data, shift, total, W_data, W_shift, W_total):
    B, L = data.shape
    info = plsc.get_sparse_core_info()
    NC, NS = info.num_cores, info.num_subcores
    NW = NC * NS

    def tr(x):
        return x.T.reshape(L, NW, BLK).transpose(1, 0, 2).astype(jnp.int32)

    out5d = _make_sc_kernel(B, L, NC, NS)(
        tr(data), tr(shift), tr(total), W_data, W_shift, W_total)
    return out5d.transpose(2, 4, 0, 1, 3).reshape(B, L, D)


# speedup vs baseline: 1.4610x; 1.4610x over previous
import functools

import jax
import jax.numpy as jnp
from jax import lax
from jax.experimental import pallas as pl
from jax.experimental.pallas import tpu as pltpu
from jax.experimental.pallas import tpu_sc as plsc

D = 64
BLK = 128


@functools.lru_cache(maxsize=None)
def _make_sc_kernel(B, L, NC, NS):
    NW = NC * NS
    KB = B // BLK
    assert KB == NW
    NBUF = 3
    n_groups = L // NBUF
    tail = L - n_groups * NBUF
    mesh = plsc.VectorSubcoreMesh(core_axis_name="c", subcore_axis_name="s")

    @functools.partial(
        pl.kernel,
        out_type=jax.ShapeDtypeStruct((L, D // 8, KB, 8, BLK), jnp.float32),
        mesh=mesh,
        compiler_params=pltpu.CompilerParams(use_tc_tiling_on_sc=False,
                                             needs_layout_passes=False),
        scratch_types=[
            pltpu.VMEM((L, BLK), jnp.int32),
            pltpu.VMEM((L, BLK), jnp.int32),
            pltpu.VMEM((L, BLK), jnp.int32),
            [pltpu.VMEM((BLK, D), jnp.float32)] * NBUF,
            [pltpu.VMEM((D // 8, 8, BLK), jnp.float32)] * NBUF,
            [pltpu.SemaphoreType.DMA] * NBUF,
            [pltpu.SemaphoreType.DMA] * NBUF,
            [pltpu.SemaphoreType.DMA] * NBUF,
        ],
    )
    def body(data_h, shift_h, total_h, wd_h, ws_h, wt_h, out_h,
             idx_d, idx_s, idx_t, accs, tbufs, gsems, asems, ssems):
        wid = lax.axis_index("s") * NC + lax.axis_index("c")
        pltpu.sync_copy(data_h.at[wid], idx_d)
        pltpu.sync_copy(shift_h.at[wid], idx_s)
        pltpu.sync_copy(total_h.at[wid], idx_t)

        lane = lax.iota(jnp.int32, 16)
        perm = {d: lane ^ d for d in (1, 2, 4, 8)}
        emask = {d: (lane & d) == 0 for d in (1, 2, 4, 8)}

        def xpose16(vs):
            for d in (1, 2, 4, 8):
                nv = list(vs)
                for i in range(16):
                    if i & d:
                        continue
                    j = i ^ d
                    a, b = vs[i], vs[j]
                    pa = a.at[perm[d]].get(mode="promise_in_bounds")
                    pb = b.at[perm[d]].get(mode="promise_in_bounds")
                    nv[i] = jnp.where(emask[d], a, pb)
                    nv[j] = jnp.where(emask[d], pa, b)
                vs = nv
            return vs

        def transpose_unit(s):
            @plsc.parallel_loop(0, BLK // 16)
            def j_body(jb):
                b0 = 16 * jb
                dst = pl.ds(b0, 16)
                for m in range(D // 16):
                    vs = [accs[s][b0 + i, pl.ds(16 * m, 16)]
                          for i in range(16)]
                    ts = xpose16(vs)
                    for f_loc in range(16):
                        f = 16 * m + f_loc
                        tbufs[s][f // 8, f % 8, dst] = ts[f_loc]

        def unit_stage1(s, l, first):
            @pl.when(jnp.logical_not(first))
            def _wait_prev_store():
                pltpu.make_async_copy(tbufs[s], out_h.at[l - NBUF, :, wid],
                                      ssems[s]).wait()
            pltpu.async_copy(wd_h.at[idx_d.at[l]], accs[s], gsems[s])

        def unit_stage2(s, l):
            pltpu.make_async_copy(wd_h.at[idx_d.at[l]], accs[s],
                                  gsems[s]).wait()
            pltpu.async_copy(ws_h.at[idx_s.at[l]], accs[s], asems[s],
                             add=True)
            pltpu.async_copy(wt_h.at[idx_t.at[l]], accs[s], asems[s],
                             add=True)

        def unit_stage3(s, l):
            add_cp = pltpu.make_async_copy(ws_h.at[idx_s.at[l]], accs[s],
                                           asems[s])
            add_cp.wait()
            add_cp.wait()
            transpose_unit(s)
            pltpu.async_copy(tbufs[s], out_h.at[l, :, wid], ssems[s])

        def group_body(g, carry):
            for s in range(NBUF):
                unit_stage1(s, g * NBUF + s, g == 0)
            for s in range(NBUF):
                unit_stage2(s, g * NBUF + s)
            for s in range(NBUF):
                unit_stage3(s, g * NBUF + s)
            return carry

        lax.fori_loop(0, n_groups, group_body, 0)
        for s in range(tail):
            l = n_groups * NBUF + s
            unit_stage1(s, l, False)
            unit_stage2(s, l)
            unit_stage3(s, l)
        for s in range(NBUF):
            l = (n_groups - 1) * NBUF + s
            if s < tail:
                l = n_groups * NBUF + s
            pltpu.make_async_copy(tbufs[s], out_h.at[l, :, wid],
                                  ssems[s]).wait()

    return body


def kernel(data, shift, total, W_data, W_shift, W_total):
    B, L = data.shape
    info = plsc.get_sparse_core_info()
    NC, NS = info.num_cores, info.num_subcores
    NW = NC * NS

    def tr(x):
        return x.T.reshape(L, NW, BLK).transpose(1, 0, 2).astype(jnp.int32)

    out5d = _make_sc_kernel(B, L, NC, NS)(
        tr(data), tr(shift), tr(total), W_data, W_shift, W_total)
    return out5d.transpose(2, 4, 0, 1, 3).reshape(B, L, D)
